# baseline (speedup 1.0000x reference)
import functools

import jax
import jax.numpy as jnp
from jax import lax
from jax.experimental import pallas as pl
from jax.experimental.pallas import tpu as pltpu
from jax.experimental.pallas import tpu_sc as plsc

B, L, D = 1024, 200, 64
NUM_LABELS = 100000

_NC, _NS = 2, 16
_NW = _NC * _NS
_RPW = B // _NW
_LA, _LB = 96, 104


def _bag_body(x_hbm, xw_hbm, table_hbm, out_hbm, idx_v, w_v, rows_v, out_v, sem):
    wid = lax.axis_index("s") * _NC + lax.axis_index("c")
    base = wid * _RPW
    pltpu.sync_copy(x_hbm.at[pl.ds(base, _RPW)], idx_v)
    pltpu.sync_copy(xw_hbm.at[pl.ds(base, _RPW)], w_v)

    def issue(r, buf):
        ca = pltpu.async_copy(
            table_hbm.at[idx_v.at[r, pl.ds(0, _LA)]],
            rows_v.at[buf, pl.ds(0, _LA)], sem)
        cb = pltpu.async_copy(
            table_hbm.at[idx_v.at[r, pl.ds(_LA, _LB)]],
            rows_v.at[buf, pl.ds(_LA, _LB)], sem)
        return ca, cb

    pending = issue(0, 0)
    for r in range(_RPW):
        buf = r % 2
        nxt = issue(r + 1, 1 - buf) if r + 1 < _RPW else None
        for c in pending:
            c.wait()

        def fma(l, w16, u, accs, _buf=buf):
            a0, a1, a2, a3 = accs
            w = w16[u]
            a0 = a0 + w * rows_v[_buf, l, pl.ds(0, 16)]
            a1 = a1 + w * rows_v[_buf, l, pl.ds(16, 16)]
            a2 = a2 + w * rows_v[_buf, l, pl.ds(32, 16)]
            a3 = a3 + w * rows_v[_buf, l, pl.ds(48, 16)]
            return a0, a1, a2, a3

        def body(i, accs, _r=r):
            w16 = w_v[_r, pl.ds(i * 16, 16)]
            for u in range(16):
                accs = fma(i * 16 + u, w16, u, accs)
            return accs

        z = jnp.zeros((16,), jnp.float32)
        accs = lax.fori_loop(0, L // 16, body, (z, z, z, z))
        w16 = w_v[r, pl.ds(L - 16, 16)]
        for u in range(8, 16):
            accs = fma(L - 16 + u, w16, u, accs)
        a0, a1, a2, a3 = accs
        out_v[r, pl.ds(0, 16)] = a0
        out_v[r, pl.ds(16, 16)] = a1
        out_v[r, pl.ds(32, 16)] = a2
        out_v[r, pl.ds(48, 16)] = a3
        if nxt is not None:
            pending = nxt
    pltpu.sync_copy(out_v, out_hbm.at[pl.ds(base, _RPW)])


_bag = functools.partial(
    pl.kernel,
    mesh=plsc.VectorSubcoreMesh(core_axis_name="c", subcore_axis_name="s"),
    compiler_params=pltpu.CompilerParams(use_tc_tiling_on_sc=False),
    out_type=jax.ShapeDtypeStruct((B, D), jnp.float32),
    scratch_types=[
        pltpu.VMEM((_RPW, L), jnp.int32),
        pltpu.VMEM((_RPW, L), jnp.float32),
        pltpu.VMEM((2, L, D), jnp.float32),
        pltpu.VMEM((_RPW, D), jnp.float32),
        pltpu.SemaphoreType.DMA,
    ],
)(_bag_body)


_BL = 2048


def _cls_body(emb_ref, wt_ref, bt_ref, wc_ref, bc_ref, out_ref, h_ref):
    @pl.when(pl.program_id(0) == 0)
    def _():
        h = jnp.dot(emb_ref[...], wt_ref[...], preferred_element_type=jnp.float32)
        h_ref[...] = jnp.maximum(h + bt_ref[...], 0.0)

    out_ref[...] = lax.dot_general(
        h_ref[...], wc_ref[...],
        dimension_numbers=(((1,), (1,)), ((), ())),
        preferred_element_type=jnp.float32,
    ) + bc_ref[...]


_classify = pl.pallas_call(
    _cls_body,
    grid=(pl.cdiv(NUM_LABELS, _BL),),
    in_specs=[
        pl.BlockSpec((B, D), lambda j: (0, 0)),
        pl.BlockSpec((D, D), lambda j: (0, 0)),
        pl.BlockSpec((1, D), lambda j: (0, 0)),
        pl.BlockSpec((_BL, D), lambda j: (j, 0)),
        pl.BlockSpec((1, _BL), lambda j: (0, j)),
    ],
    out_specs=pl.BlockSpec((B, _BL), lambda j: (0, j)),
    out_shape=jax.ShapeDtypeStruct((B, NUM_LABELS), jnp.float32),
    scratch_shapes=[pltpu.VMEM((B, D), jnp.float32)],
)


def kernel(X, X_w, emb_table, W_t, b_t, W_c, b_c):
    embed = _bag(X, X_w, emb_table)
    return _classify(embed, W_t, b_t.reshape(1, D), W_c, b_c.reshape(1, NUM_LABELS))
